# baseline (speedup 1.0000x reference)
import functools

import jax
import jax.numpy as jnp
from jax import lax
from jax.experimental import pallas as pl
from jax.experimental.pallas import tpu as pltpu
from jax.experimental.pallas import tpu_sc as plsc

NUM_EMB = 100000
D = 128
B, HL = 4096, 50
NC, NS = 2, 16
NW = NC * NS
PER_W = B // NW
G = 8
GR = G * HL
N_IT = PER_W // G

_mesh = plsc.VectorSubcoreMesh(core_axis_name="c", subcore_axis_name="s")


@functools.partial(
    pl.kernel,
    mesh=_mesh,
    out_type=jax.ShapeDtypeStruct((B * HL, D), jnp.float32),
    scratch_types=[
        pltpu.VMEM((2, G, HL), jnp.int32),
        pltpu.VMEM((2, GR, D), jnp.float32),
        pltpu.SemaphoreType.DMA,
        pltpu.SemaphoreType.DMA,
        pltpu.SemaphoreType.DMA,
    ],
)
def _sc_gather(table_hbm, x_hbm, out_hbm, idx_v, rows_v, sem_i, sem_g, sem_o):
    wid = lax.axis_index("s") * NC + lax.axis_index("c")
    base = wid * PER_W

    pltpu.async_copy(x_hbm.at[pl.ds(base, G)], idx_v.at[0], sem_i)

    def pair_body(p, carry):
        for b in range(2):
            i = p * 2 + b
            r0 = base + i * G

            @pl.when(i >= 2)
            def _():
                pltpu.make_async_copy(
                    rows_v.at[b],
                    out_hbm.at[pl.ds((r0 - 2 * G) * HL, GR)], sem_o,
                ).wait()

            pltpu.make_async_copy(
                x_hbm.at[pl.ds(r0, G)], idx_v.at[b], sem_i).wait()

            @pl.when(i + 1 < N_IT)
            def _():
                pltpu.async_copy(
                    x_hbm.at[pl.ds(r0 + G, G)], idx_v.at[1 - b], sem_i)

            copies = [
                pltpu.async_copy(
                    table_hbm.at[idx_v.at[b, j]],
                    rows_v.at[b, pl.ds(j * HL, HL)], sem_g)
                for j in range(G)
            ]
            for cp in copies:
                cp.wait()

            pltpu.async_copy(
                rows_v.at[b], out_hbm.at[pl.ds(r0 * HL, GR)], sem_o)
        return carry

    lax.fori_loop(0, N_IT // 2, pair_body, 0)

    pltpu.make_async_copy(
        rows_v.at[0],
        out_hbm.at[pl.ds((base + (N_IT - 2) * G) * HL, GR)], sem_o,
    ).wait()
    pltpu.make_async_copy(
        rows_v.at[1],
        out_hbm.at[pl.ds((base + (N_IT - 1) * G) * HL, GR)], sem_o,
    ).wait()


TROWS_B = 4000


def _dequant_body(cinv_ref, in_ref, out_ref):
    out_ref[...] = in_ref[...].astype(jnp.float32) * cinv_ref[0]


_dequant_table = pl.pallas_call(
    _dequant_body,
    grid=(NUM_EMB // TROWS_B,),
    in_specs=[
        pl.BlockSpec(memory_space=pltpu.SMEM),
        pl.BlockSpec((TROWS_B, D), lambda i: (i, 0)),
    ],
    out_specs=pl.BlockSpec((TROWS_B, D), lambda i: (i, 0)),
    out_shape=jax.ShapeDtypeStruct((NUM_EMB, D), jnp.float32),
)


RB = 8


def _retile_body(in_ref, out_ref):
    out_ref[...] = in_ref[...].reshape(RB, HL, D)


_retile = pl.pallas_call(
    _retile_body,
    grid=(B // RB,),
    in_specs=[pl.BlockSpec((RB * HL, D), lambda i: (i, 0))],
    out_specs=pl.BlockSpec((RB, HL, D), lambda i: (i, 0, 0)),
    out_shape=jax.ShapeDtypeStruct((B, HL, D), jnp.float32),
)


def kernel(x, weight_quant, c):
    cinv = (jnp.float32(1.0) / c.astype(jnp.float32)).reshape(1)
    table_f32 = _dequant_table(cinv, weight_quant)
    flat = _sc_gather(table_f32, x.astype(jnp.int32))
    return _retile(flat)


# speedup vs baseline: 2.7103x; 2.7103x over previous
import functools

import jax
import jax.numpy as jnp
from jax import lax
from jax.experimental import pallas as pl
from jax.experimental.pallas import tpu as pltpu
from jax.experimental.pallas import tpu_sc as plsc

NUM_EMB = 100000
D = 128
B, HL = 4096, 50
NC, NS = 2, 16
NW = NC * NS
PER_W = B // NW
G = 8
N_IT = PER_W // G

_mesh = plsc.VectorSubcoreMesh(core_axis_name="c", subcore_axis_name="s")


@functools.partial(
    pl.kernel,
    mesh=_mesh,
    out_type=jax.ShapeDtypeStruct((B, HL, D), jnp.float32),
    scratch_types=[
        pltpu.VMEM((2, G, HL), jnp.int32),
        pltpu.VMEM((2, G, HL, D), jnp.float32),
        pltpu.SemaphoreType.DMA,
        pltpu.SemaphoreType.DMA,
        pltpu.SemaphoreType.DMA,
    ],
)
def _sc_gather(table_hbm, x_hbm, out_hbm, idx_v, rows_v, sem_i, sem_g, sem_o):
    wid = lax.axis_index("s") * NC + lax.axis_index("c")
    base = wid * PER_W

    pltpu.async_copy(x_hbm.at[pl.ds(base, G)], idx_v.at[0], sem_i)

    def pair_body(p, carry):
        for b in range(2):
            i = p * 2 + b
            r0 = base + i * G

            @pl.when(i >= 2)
            def _():
                pltpu.make_async_copy(
                    rows_v.at[b], out_hbm.at[pl.ds(r0 - 2 * G, G)], sem_o
                ).wait()

            pltpu.make_async_copy(
                x_hbm.at[pl.ds(r0, G)], idx_v.at[b], sem_i).wait()

            @pl.when(i + 1 < N_IT)
            def _():
                pltpu.async_copy(
                    x_hbm.at[pl.ds(r0 + G, G)], idx_v.at[1 - b], sem_i)

            copies = [
                pltpu.async_copy(
                    table_hbm.at[idx_v.at[b, j]], rows_v.at[b, j], sem_g)
                for j in range(G)
            ]
            for cp in copies:
                cp.wait()

            pltpu.async_copy(rows_v.at[b], out_hbm.at[pl.ds(r0, G)], sem_o)
        return carry

    lax.fori_loop(0, N_IT // 2, pair_body, 0)

    pltpu.make_async_copy(
        rows_v.at[0], out_hbm.at[pl.ds(base + (N_IT - 2) * G, G)], sem_o
    ).wait()
    pltpu.make_async_copy(
        rows_v.at[1], out_hbm.at[pl.ds(base + (N_IT - 1) * G, G)], sem_o
    ).wait()


TROWS_B = 20000


def _dequant_body(cinv_ref, in_ref, out_ref):
    out_ref[...] = in_ref[...].astype(jnp.float32) * cinv_ref[0]


_dequant_table = pl.pallas_call(
    _dequant_body,
    grid=(NUM_EMB // TROWS_B,),
    in_specs=[
        pl.BlockSpec(memory_space=pltpu.SMEM),
        pl.BlockSpec((TROWS_B, D), lambda i: (i, 0)),
    ],
    out_specs=pl.BlockSpec((TROWS_B, D), lambda i: (i, 0)),
    out_shape=jax.ShapeDtypeStruct((NUM_EMB, D), jnp.float32),
)


def kernel(x, weight_quant, c):
    cinv = (jnp.float32(1.0) / c.astype(jnp.float32)).reshape(1)
    table_f32 = _dequant_table(cinv, weight_quant)
    return _sc_gather(table_f32, x.astype(jnp.int32))
